# baseline (speedup 1.0000x reference)
import functools

import jax
import jax.numpy as jnp
from jax import lax
from jax.experimental import pallas as pl
from jax.experimental.pallas import tpu as pltpu
from jax.experimental.pallas import tpu_sc as plsc

_IB = 1408


def _ffn_kernel(x_ref, w1_ref, w3_ref, w2_ref, out_ref, acc_ref):
    j = pl.program_id(1)
    nj = pl.num_programs(1)

    xv = x_ref[...]
    g = jax.lax.dot_general(xv, w1_ref[...], (((1,), (1,)), ((), ())),
                            preferred_element_type=jnp.float32)
    u = jax.lax.dot_general(xv, w3_ref[...], (((1,), (1,)), ((), ())),
                            preferred_element_type=jnp.float32)
    h = g * jax.nn.sigmoid(g) * u
    pe = jax.lax.dot_general(h, w2_ref[...], (((1,), (1,)), ((), ())),
                             preferred_element_type=jnp.float32)

    @pl.when(j == 0)
    def _():
        acc_ref[...] = pe

    @pl.when((j > 0) & (j < nj - 1))
    def _():
        acc_ref[...] += pe

    @pl.when(j == nj - 1)
    def _():
        out_ref[...] = acc_ref[...] + pe


def _all_experts_ffn(x, w1, w2, w3):
    E, I, D = w1.shape
    T = x.shape[0]
    nj = I // _IB
    return pl.pallas_call(
        _ffn_kernel,
        grid=(E, nj),
        in_specs=[
            pl.BlockSpec((T, D), lambda e, j: (0, 0)),
            pl.BlockSpec((None, _IB, D), lambda e, j: (e, j, 0)),
            pl.BlockSpec((None, _IB, D), lambda e, j: (e, j, 0)),
            pl.BlockSpec((None, D, _IB), lambda e, j: (e, 0, j)),
        ],
        out_specs=pl.BlockSpec((T, D), lambda e, j: (e, 0)),
        out_shape=jax.ShapeDtypeStruct((E * T, D), jnp.float32),
        scratch_shapes=[pltpu.VMEM((T, D), jnp.float32)],
        compiler_params=pltpu.CompilerParams(
            dimension_semantics=("arbitrary", "arbitrary"),
        ),
    )(x, w1, w3, w2)


def _sc_route(idx_flat, table, T, A, D):
    info = plsc.get_sparse_core_info()
    NC = info.num_cores
    B = idx_flat.shape[0]
    per_w = 16
    n_active = B // per_w
    mesh = plsc.VectorSubcoreMesh(core_axis_name="c", subcore_axis_name="s")

    @functools.partial(
        pl.kernel, mesh=mesh,
        out_type=jax.ShapeDtypeStruct((B, D), jnp.float32),
        scratch_types=[
            pltpu.VMEM((per_w,), jnp.int32),
            pltpu.VMEM((per_w,), jnp.int32),
            pltpu.VMEM((per_w, D), jnp.float32),
            pltpu.SemaphoreType.DMA,
        ],
    )
    def gather_k(idx_hbm, table_hbm, out_hbm, idx_v, r_v, rows_v, sem):
        wid = lax.axis_index("s") * NC + lax.axis_index("c")

        @pl.when(wid < n_active)
        def _():
            base = wid * per_w
            pltpu.sync_copy(idx_hbm.at[pl.ds(base, per_w)], idx_v)
            i = lax.iota(jnp.int32, per_w)
            t = base // A + lax.div(i, jnp.int32(A))
            r_v[...] = idx_v[...] * T + t
            pltpu.async_copy(table_hbm.at[r_v], rows_v, sem).wait()
            pltpu.sync_copy(rows_v, out_hbm.at[pl.ds(base, per_w)])

    return gather_k(idx_flat, table)


def kernel(x, expert_indices, w1, w2, w3):
    E, I, D = w1.shape
    T = x.shape[0]
    A = expert_indices.shape[1]
    idx_flat = expert_indices.astype(jnp.int32).reshape(-1)

    table = _all_experts_ffn(x, w1, w2, w3)
    routed = _sc_route(idx_flat, table, T, A, D)
    return routed.reshape(T, A, D)


# speedup vs baseline: 1.0135x; 1.0135x over previous
import functools

import jax
import jax.numpy as jnp
from jax import lax
from jax.experimental import pallas as pl
from jax.experimental.pallas import tpu as pltpu
from jax.experimental.pallas import tpu_sc as plsc

_IB = 1408


def _ffn_kernel(x_ref, w1_ref, w3_ref, w2_ref, out_ref, acc_ref):
    j = pl.program_id(1)
    nj = pl.num_programs(1)

    xv = x_ref[...]
    g = jax.lax.dot_general(xv, w1_ref[...], (((1,), (1,)), ((), ())),
                            preferred_element_type=jnp.float32)
    u = jax.lax.dot_general(xv, w3_ref[...], (((1,), (1,)), ((), ())),
                            preferred_element_type=jnp.float32)
    h = g * jax.nn.sigmoid(g) * u
    pe = jax.lax.dot_general(h, w2_ref[...], (((1,), (1,)), ((), ())),
                             preferred_element_type=jnp.float32)

    @pl.when(j == 0)
    def _():
        acc_ref[...] = pe

    @pl.when((j > 0) & (j < nj - 1))
    def _():
        acc_ref[...] += pe

    @pl.when(j == nj - 1)
    def _():
        out_ref[...] = acc_ref[...] + pe


def _all_experts_ffn(x, w1, w2, w3):
    E, I, D = w1.shape
    T = x.shape[0]
    nj = I // _IB
    return pl.pallas_call(
        _ffn_kernel,
        grid=(E, nj),
        in_specs=[
            pl.BlockSpec((T, D), lambda e, j: (0, 0)),
            pl.BlockSpec((None, _IB, D), lambda e, j: (e, j, 0)),
            pl.BlockSpec((None, _IB, D), lambda e, j: (e, j, 0)),
            pl.BlockSpec((None, D, _IB), lambda e, j: (e, 0, j)),
        ],
        out_specs=pl.BlockSpec((T, D), lambda e, j: (e, 0)),
        out_shape=jax.ShapeDtypeStruct((E * T, D), jnp.float32),
        scratch_shapes=[pltpu.VMEM((T, D), jnp.float32)],
        compiler_params=pltpu.CompilerParams(
            dimension_semantics=("arbitrary", "arbitrary"),
        ),
    )(x, w1, w3, w2)


def _sc_route(idx_flat, table, T, A, D):
    NC = 1
    B = idx_flat.shape[0]
    per_w = 16
    n_active = B // per_w
    mesh = plsc.VectorSubcoreMesh(core_axis_name="c", subcore_axis_name="s",
                                  num_cores=1)

    @functools.partial(
        pl.kernel, mesh=mesh,
        out_type=jax.ShapeDtypeStruct((B, D), jnp.float32),
        scratch_types=[
            pltpu.VMEM((per_w,), jnp.int32),
            pltpu.VMEM((per_w,), jnp.int32),
            pltpu.VMEM((per_w, D), jnp.float32),
            pltpu.SemaphoreType.DMA,
        ],
    )
    def gather_k(idx_hbm, table_hbm, out_hbm, idx_v, r_v, rows_v, sem):
        wid = lax.axis_index("s") * NC + lax.axis_index("c")

        @pl.when(wid < n_active)
        def _():
            base = wid * per_w
            pltpu.sync_copy(idx_hbm.at[pl.ds(base, per_w)], idx_v)
            i = lax.iota(jnp.int32, per_w)
            t = base // A + lax.div(i, jnp.int32(A))
            r_v[...] = idx_v[...] * T + t
            pltpu.async_copy(table_hbm.at[r_v], rows_v, sem).wait()
            pltpu.sync_copy(rows_v, out_hbm.at[pl.ds(base, per_w)])

    return gather_k(idx_flat, table)


def kernel(x, expert_indices, w1, w2, w3):
    E, I, D = w1.shape
    T = x.shape[0]
    A = expert_indices.shape[1]
    idx_flat = expert_indices.astype(jnp.int32).reshape(-1)

    table = _all_experts_ffn(x, w1, w2, w3)
    routed = _sc_route(idx_flat, table, T, A, D)
    return routed.reshape(T, A, D)


# speedup vs baseline: 1.0222x; 1.0086x over previous
import functools

import jax
import jax.numpy as jnp
from jax import lax
from jax.experimental import pallas as pl
from jax.experimental.pallas import tpu as pltpu
from jax.experimental.pallas import tpu_sc as plsc

_IB = 1408


def _ffn_kernel(x_ref, w1_ref, w3_ref, w2_ref, out_ref, acc_ref):
    j = pl.program_id(1)
    nj = pl.num_programs(1)

    xv = x_ref[...]
    g = jax.lax.dot_general(xv, w1_ref[...], (((1,), (1,)), ((), ())),
                            preferred_element_type=jnp.float32)
    u = jax.lax.dot_general(xv, w3_ref[...], (((1,), (1,)), ((), ())),
                            preferred_element_type=jnp.float32)
    h = g * jax.nn.sigmoid(g) * u
    pe = jax.lax.dot_general(h, w2_ref[...], (((1,), (1,)), ((), ())),
                             preferred_element_type=jnp.float32)

    @pl.when(j == 0)
    def _():
        acc_ref[...] = pe

    @pl.when((j > 0) & (j < nj - 1))
    def _():
        acc_ref[...] += pe

    @pl.when(j == nj - 1)
    def _():
        out_ref[...] = acc_ref[...] + pe


def _all_experts_ffn(x, w1, w2, w3):
    E, I, D = w1.shape
    T = x.shape[0]
    nj = I // _IB
    return pl.pallas_call(
        _ffn_kernel,
        grid=(E, nj),
        in_specs=[
            pl.BlockSpec((T, D), lambda e, j: (0, 0)),
            pl.BlockSpec((None, _IB, D), lambda e, j: (e, j, 0)),
            pl.BlockSpec((None, _IB, D), lambda e, j: (e, j, 0)),
            pl.BlockSpec((None, D, _IB), lambda e, j: (e, 0, j)),
        ],
        out_specs=pl.BlockSpec((T, D), lambda e, j: (e, 0)),
        out_shape=jax.ShapeDtypeStruct((E * T, D), jnp.float32),
        scratch_shapes=[pltpu.VMEM((T, D), jnp.float32)],
        compiler_params=pltpu.CompilerParams(
            dimension_semantics=("arbitrary", "arbitrary"),
        ),
    )(x, w1, w3, w2)


def _sc_route(idx_flat, table, T, A, D):
    B = idx_flat.shape[0]
    chunk = 16
    half = chunk // 2
    n_active = B // half
    mesh = plsc.VectorSubcoreMesh(core_axis_name="c", subcore_axis_name="s",
                                  num_cores=1)

    @functools.partial(
        pl.kernel, mesh=mesh,
        out_type=jax.ShapeDtypeStruct((B, D), jnp.float32),
        scratch_types=[
            pltpu.VMEM((chunk,), jnp.int32),
            pltpu.VMEM((chunk,), jnp.int32),
            pltpu.VMEM((half, D), jnp.float32),
            pltpu.SemaphoreType.DMA,
        ],
    )
    def gather_k(idx_hbm, table_hbm, out_hbm, idx_v, r_v, rows_v, sem):
        wid = lax.axis_index("s")

        @pl.when(wid < n_active)
        def _():
            cbase = (wid // 2) * chunk
            pltpu.sync_copy(idx_hbm.at[pl.ds(cbase, chunk)], idx_v)
            i = lax.iota(jnp.int32, chunk)
            t = cbase // A + lax.div(i, jnp.int32(A))
            r_v[...] = idx_v[...] * T + t
            pltpu.async_copy(
                table_hbm.at[r_v.at[pl.ds((wid % 2) * half, half)]],
                rows_v, sem).wait()
            pltpu.sync_copy(rows_v, out_hbm.at[pl.ds(wid * half, half)])

    return gather_k(idx_flat, table)


def kernel(x, expert_indices, w1, w2, w3):
    E, I, D = w1.shape
    T = x.shape[0]
    A = expert_indices.shape[1]
    idx_flat = expert_indices.astype(jnp.int32).reshape(-1)

    table = _all_experts_ffn(x, w1, w2, w3)
    routed = _sc_route(idx_flat, table, T, A, D)
    return routed.reshape(T, A, D)
